# baseline (speedup 1.0000x reference)
import functools
import math

import jax
import jax.numpy as jnp
import numpy as np
from jax import lax
from jax.experimental import pallas as pl
from jax.experimental.pallas import tpu as pltpu
from jax.experimental.pallas import tpu_sc as plsc

_NUM_CORES = 2
_NUM_SUBCORES = 16
_NCHUNK = 4


def _positional_encoding(d_model, length):
    position = np.arange(length, dtype=np.float32)[:, None]
    div_term = np.exp(
        np.arange(0, d_model, 2, dtype=np.float32) * (-math.log(10000.0) / d_model))
    pe = np.zeros((length, d_model), dtype=np.float32)
    pe[:, 0::2] = np.sin(position * div_term)
    pe[:, 1::2] = np.cos(position * div_term)
    return jnp.asarray(pe)


def _tc_inds(align2d, alignm1, B, T):
    R = B * 32
    U = jnp.asarray(np.triu(np.ones((128, 128), np.float32)))
    O = jnp.asarray(np.ones((128, 128), np.float32))
    Lb = jnp.asarray(np.kron(np.eye(B, dtype=np.float32),
                             np.tril(np.ones((32, 32), np.float32), -1)))
    base = jnp.asarray(
        np.repeat(np.arange(B, dtype=np.int32) * T, 32)[:, None]
        * np.ones((1, 128), np.int32))

    def body(a_ref, p_ref, u_ref, o_ref, l_ref, b_ref, out_ref):
        flags = jnp.where(a_ref[...] != p_ref[...], 1.0, 0.0)
        s1 = jnp.dot(flags, u_ref[...], preferred_element_type=jnp.float32)
        rs = jnp.dot(flags, o_ref[...], preferred_element_type=jnp.float32)
        off = jnp.dot(l_ref[...], rs, preferred_element_type=jnp.float32)
        out_ref[...] = (s1 + off).astype(jnp.int32) + b_ref[...]

    out = pl.pallas_call(
        body,
        out_shape=jax.ShapeDtypeStruct((R, 128), jnp.int32),
    )(align2d, alignm1, U, O, Lb, base)
    return out.reshape(B * T)


def _sc_gather_chunk(enc_flat, inds, k, B, T, D):
    NW = _NUM_CORES * _NUM_SUBCORES
    rows = (B // _NCHUNK) * T
    span = rows // NW
    CH = 128

    mesh = plsc.VectorSubcoreMesh(core_axis_name="c", subcore_axis_name="s")

    @functools.partial(
        pl.kernel,
        out_type=jax.ShapeDtypeStruct((rows, D), jnp.float32),
        mesh=mesh,
        scratch_types=[
            pltpu.VMEM((span,), jnp.int32),
            pltpu.VMEM((CH, D), jnp.float32),
            pltpu.VMEM((CH, D), jnp.float32),
            pltpu.SemaphoreType.DMA,
            pltpu.SemaphoreType.DMA,
        ],
        compiler_params=pltpu.CompilerParams(needs_layout_passes=False),
    )
    def sc_kernel(enc_hbm, inds_hbm, out_hbm, idx_v, rows_a, rows_b,
                  sem_a, sem_b):
        wid = lax.axis_index("s") * _NUM_CORES + lax.axis_index("c")
        local0 = wid * span
        pltpu.sync_copy(inds_hbm.at[pl.ds(k * rows + local0, span)], idx_v)

        def gather_pair(j2, carry):
            off_a = (2 * j2) * CH
            off_b = off_a + CH
            da = pltpu.async_copy(enc_hbm.at[idx_v.at[pl.ds(off_a, CH)]],
                                  rows_a, sem_a)
            db = pltpu.async_copy(enc_hbm.at[idx_v.at[pl.ds(off_b, CH)]],
                                  rows_b, sem_b)
            da.wait()
            pltpu.sync_copy(rows_a, out_hbm.at[pl.ds(local0 + off_a, CH)])
            db.wait()
            pltpu.sync_copy(rows_b, out_hbm.at[pl.ds(local0 + off_b, CH)])
            return carry

        lax.fori_loop(0, span // (2 * CH), gather_pair, jnp.int32(0))

    return sc_kernel(enc_flat, inds)


def _tc_postnet_chunk(gk, pe, pitch_k, beats_k, W2, wp, dvec, cvec,
                      prev, k, B, T, D):
    nb = B // _NCHUNK

    def body(*refs):
        g_ref, pe_ref, p_ref, bt_ref, w2_ref, wp_ref, dv_ref, cv_ref = refs[:8]
        o_ref = refs[-1]
        pe_t = pe_ref[...]
        x = g_ref[0] + pe_t
        acc = jnp.dot(x, w2_ref[...], preferred_element_type=jnp.float32)
        bt = bt_ref[0].astype(jnp.float32)
        o_ref[0] = (acc - pe_t + p_ref[0] * wp_ref[...]
                    + bt * dv_ref[...] + cv_ref[...])

    in_specs = [
        pl.BlockSpec((1, T, D), lambda bb: (bb, 0, 0)),
        pl.BlockSpec((T, D), lambda bb: (0, 0)),
        pl.BlockSpec((1, T, 1), lambda bb: (bb, 0, 0)),
        pl.BlockSpec((1, T, 1), lambda bb: (bb, 0, 0)),
        pl.BlockSpec((D, D), lambda bb: (0, 0)),
        pl.BlockSpec((1, D), lambda bb: (0, 0)),
        pl.BlockSpec((1, D), lambda bb: (0, 0)),
        pl.BlockSpec((1, D), lambda bb: (0, 0)),
    ]
    args = [gk.reshape(nb, T, D), pe, pitch_k, beats_k, W2, wp, dvec, cvec]
    aliases = {}
    if prev is not None:
        in_specs.append(pl.BlockSpec(memory_space=pl.ANY))
        args.append(prev)
        aliases = {8: 0}

    return pl.pallas_call(
        body,
        grid=(nb,),
        in_specs=in_specs,
        out_specs=pl.BlockSpec((1, T, D), lambda bb, _k=k: (_k * nb + bb, 0, 0)),
        out_shape=jax.ShapeDtypeStruct((B, T, D), jnp.float32),
        input_output_aliases=aliases,
    )(*args)


def kernel(encoder_out, align_phone, text_phone, pitch, beats,
           W_pitch, b_pitch, W_pos, b_pos, emb_beats):
    del text_phone
    B, T, D = encoder_out.shape
    enc_flat = encoder_out.reshape(B * T, D)

    align2d = align_phone.reshape(B * 32, 128)
    alignm1 = jnp.concatenate(
        [align_phone[:, :1], align_phone[:, :-1]], axis=1).reshape(B * 32, 128)
    inds = _tc_inds(align2d, alignm1, B, T)

    pe = _positional_encoding(D, T)
    W2 = W_pos.T + jnp.eye(D, dtype=W_pos.dtype)
    wp = W_pitch.reshape(1, D)
    dvec = (emb_beats[1] - emb_beats[0]).reshape(1, D)
    cvec = (b_pos + b_pitch + emb_beats[0]).reshape(1, D)

    nb = B // _NCHUNK
    out = None
    for k in range(_NCHUNK):
        gk = _sc_gather_chunk(enc_flat, inds, k, B, T, D)
        pitch_k = lax.slice_in_dim(pitch, k * nb, (k + 1) * nb, axis=0)
        beats_k = lax.slice_in_dim(beats, k * nb, (k + 1) * nb, axis=0)
        out = _tc_postnet_chunk(gk, pe, pitch_k, beats_k, W2, wp, dvec, cvec,
                                out, k, B, T, D)
    return out


# speedup vs baseline: 1.0009x; 1.0009x over previous
import functools
import math

import jax
import jax.numpy as jnp
import numpy as np
from jax import lax
from jax.experimental import pallas as pl
from jax.experimental.pallas import tpu as pltpu
from jax.experimental.pallas import tpu_sc as plsc

_NUM_CORES = 2
_NUM_SUBCORES = 16
_NCHUNK = 4


def _positional_encoding(d_model, length):
    position = np.arange(length, dtype=np.float32)[:, None]
    div_term = np.exp(
        np.arange(0, d_model, 2, dtype=np.float32) * (-math.log(10000.0) / d_model))
    pe = np.zeros((length, d_model), dtype=np.float32)
    pe[:, 0::2] = np.sin(position * div_term)
    pe[:, 1::2] = np.cos(position * div_term)
    return jnp.asarray(pe)


def _tc_inds(align2d, alignm1, B, T):
    R = B * 32
    U = jnp.asarray(np.triu(np.ones((128, 128), np.float32)))
    O = jnp.asarray(np.ones((128, 128), np.float32))
    Lb = jnp.asarray(np.kron(np.eye(B, dtype=np.float32),
                             np.tril(np.ones((32, 32), np.float32), -1)))
    base = jnp.asarray(
        np.repeat(np.arange(B, dtype=np.int32) * T, 32)[:, None]
        * np.ones((1, 128), np.int32))

    def body(a_ref, p_ref, u_ref, o_ref, l_ref, b_ref, out_ref):
        flags = jnp.where(a_ref[...] != p_ref[...], 1.0, 0.0)
        s1 = jnp.dot(flags, u_ref[...], preferred_element_type=jnp.float32)
        rs = jnp.dot(flags, o_ref[...], preferred_element_type=jnp.float32)
        off = jnp.dot(l_ref[...], rs, preferred_element_type=jnp.float32)
        out_ref[...] = (s1 + off).astype(jnp.int32) + b_ref[...]

    out = pl.pallas_call(
        body,
        out_shape=jax.ShapeDtypeStruct((R, 128), jnp.int32),
    )(align2d, alignm1, U, O, Lb, base)
    return out.reshape(B * T)


def _sc_gather_chunk(enc_flat, inds, k, B, T, D):
    NW = _NUM_CORES * _NUM_SUBCORES
    rows = (B // _NCHUNK) * T
    span = rows // NW
    CH = 128

    mesh = plsc.VectorSubcoreMesh(core_axis_name="c", subcore_axis_name="s")

    @functools.partial(
        pl.kernel,
        out_type=jax.ShapeDtypeStruct((rows, D), jnp.float32),
        mesh=mesh,
        scratch_types=[
            pltpu.VMEM((span,), jnp.int32),
            pltpu.VMEM((CH, D), jnp.float32),
            pltpu.VMEM((CH, D), jnp.float32),
            pltpu.SemaphoreType.DMA,
            pltpu.SemaphoreType.DMA,
        ],
        compiler_params=pltpu.CompilerParams(needs_layout_passes=False),
    )
    def sc_kernel(enc_hbm, inds_hbm, out_hbm, idx_v, rows_a, rows_b,
                  sem_a, sem_b):
        wid = lax.axis_index("s") * _NUM_CORES + lax.axis_index("c")
        local0 = wid * span
        pltpu.sync_copy(inds_hbm.at[pl.ds(k * rows + local0, span)], idx_v)

        def gather_pair(j2, carry):
            off_a = (2 * j2) * CH
            off_b = off_a + CH
            da = pltpu.async_copy(enc_hbm.at[idx_v.at[pl.ds(off_a, CH)]],
                                  rows_a, sem_a)
            db = pltpu.async_copy(enc_hbm.at[idx_v.at[pl.ds(off_b, CH)]],
                                  rows_b, sem_b)
            da.wait()
            pltpu.sync_copy(rows_a, out_hbm.at[pl.ds(local0 + off_a, CH)])
            db.wait()
            pltpu.sync_copy(rows_b, out_hbm.at[pl.ds(local0 + off_b, CH)])
            return carry

        lax.fori_loop(0, span // (2 * CH), gather_pair, jnp.int32(0))

    return sc_kernel(enc_flat, inds)


def _tc_postnet_chunk(gk, pe, pitch_k, beats_k, W2, wp, dvec, cvec,
                      prev, k, B, T, D):
    nb = B // _NCHUNK

    def body(*refs):
        g_ref, pe_ref, p_ref, bt_ref, w2_ref, wp_ref, dv_ref, cv_ref = refs[:8]
        o_ref = refs[-1]
        pe_t = pe_ref[...]
        x = g_ref[0] + pe_t
        acc = jnp.dot(x, w2_ref[...], preferred_element_type=jnp.float32)
        bt = bt_ref[0].astype(jnp.float32)
        o_ref[0] = (acc - pe_t + p_ref[0] * wp_ref[...]
                    + bt * dv_ref[...] + cv_ref[...])

    in_specs = [
        pl.BlockSpec((1, T, D), lambda bb: (bb, 0, 0)),
        pl.BlockSpec((T, D), lambda bb: (0, 0)),
        pl.BlockSpec((1, T, 1), lambda bb: (bb, 0, 0)),
        pl.BlockSpec((1, T, 1), lambda bb: (bb, 0, 0)),
        pl.BlockSpec((D, D), lambda bb: (0, 0)),
        pl.BlockSpec((1, D), lambda bb: (0, 0)),
        pl.BlockSpec((1, D), lambda bb: (0, 0)),
        pl.BlockSpec((1, D), lambda bb: (0, 0)),
    ]
    args = [gk.reshape(nb, T, D), pe, pitch_k, beats_k, W2, wp, dvec, cvec]
    aliases = {}
    if prev is not None:
        in_specs.append(pl.BlockSpec(memory_space=pl.ANY))
        args.append(prev)
        aliases = {8: 0}

    return pl.pallas_call(
        body,
        grid=(nb,),
        in_specs=in_specs,
        out_specs=pl.BlockSpec((1, T, D), lambda bb, _k=k: (_k * nb + bb, 0, 0)),
        out_shape=jax.ShapeDtypeStruct((B, T, D), jnp.float32),
        input_output_aliases=aliases,
    )(*args)


def kernel(encoder_out, align_phone, text_phone, pitch, beats,
           W_pitch, b_pitch, W_pos, b_pos, emb_beats):
    del text_phone
    B, T, D = encoder_out.shape
    enc_flat = encoder_out.reshape(B * T, D)

    align2d = align_phone.reshape(B * 32, 128)
    alignm1 = jnp.concatenate(
        [align_phone[:, :1], align_phone[:, :-1]], axis=1).reshape(B * 32, 128)
    inds = _tc_inds(align2d, alignm1, B, T)

    pe = _positional_encoding(D, T)
    W2 = W_pos.T + jnp.eye(D, dtype=W_pos.dtype)
    wp = W_pitch.reshape(1, D)
    dvec = (emb_beats[1] - emb_beats[0]).reshape(1, D)
    cvec = (b_pos + b_pitch + emb_beats[0]).reshape(1, D)

    nb = B // _NCHUNK
    gs = [_sc_gather_chunk(enc_flat, inds, k, B, T, D)
          for k in range(_NCHUNK)]
    out = None
    for k in range(_NCHUNK):
        pitch_k = lax.slice_in_dim(pitch, k * nb, (k + 1) * nb, axis=0)
        beats_k = lax.slice_in_dim(beats, k * nb, (k + 1) * nb, axis=0)
        out = _tc_postnet_chunk(gs[k], pe, pitch_k, beats_k, W2, wp, dvec,
                                cvec, out, k, B, T, D)
    return out


# speedup vs baseline: 1.0855x; 1.0845x over previous
import functools
import math

import jax
import jax.numpy as jnp
import numpy as np
from jax import lax
from jax.experimental import pallas as pl
from jax.experimental.pallas import tpu as pltpu
from jax.experimental.pallas import tpu_sc as plsc

_NUM_CORES = 2
_NUM_SUBCORES = 16
_NCHUNK = 1


def _positional_encoding(d_model, length):
    position = np.arange(length, dtype=np.float32)[:, None]
    div_term = np.exp(
        np.arange(0, d_model, 2, dtype=np.float32) * (-math.log(10000.0) / d_model))
    pe = np.zeros((length, d_model), dtype=np.float32)
    pe[:, 0::2] = np.sin(position * div_term)
    pe[:, 1::2] = np.cos(position * div_term)
    return jnp.asarray(pe)


def _tc_inds(align2d, alignm1, B, T):
    R = B * 32
    U = jnp.asarray(np.triu(np.ones((128, 128), np.float32)))
    O = jnp.asarray(np.ones((128, 128), np.float32))
    Lb = jnp.asarray(np.kron(np.eye(B, dtype=np.float32),
                             np.tril(np.ones((32, 32), np.float32), -1)))
    base = jnp.asarray(
        np.repeat(np.arange(B, dtype=np.int32) * T, 32)[:, None]
        * np.ones((1, 128), np.int32))

    def body(a_ref, p_ref, u_ref, o_ref, l_ref, b_ref, out_ref):
        flags = jnp.where(a_ref[...] != p_ref[...], 1.0, 0.0)
        s1 = jnp.dot(flags, u_ref[...], preferred_element_type=jnp.float32)
        rs = jnp.dot(flags, o_ref[...], preferred_element_type=jnp.float32)
        off = jnp.dot(l_ref[...], rs, preferred_element_type=jnp.float32)
        out_ref[...] = (s1 + off).astype(jnp.int32) + b_ref[...]

    out = pl.pallas_call(
        body,
        out_shape=jax.ShapeDtypeStruct((R, 128), jnp.int32),
    )(align2d, alignm1, U, O, Lb, base)
    return out.reshape(B * T)


def _sc_gather_chunk(enc_flat, inds, k, B, T, D):
    NW = _NUM_CORES * _NUM_SUBCORES
    rows = (B // _NCHUNK) * T
    span = rows // NW
    CH = 128

    mesh = plsc.VectorSubcoreMesh(core_axis_name="c", subcore_axis_name="s")

    @functools.partial(
        pl.kernel,
        out_type=jax.ShapeDtypeStruct((rows, D), jnp.float32),
        mesh=mesh,
        scratch_types=[
            pltpu.VMEM((span,), jnp.int32),
            pltpu.VMEM((CH, D), jnp.float32),
            pltpu.VMEM((CH, D), jnp.float32),
            pltpu.SemaphoreType.DMA,
            pltpu.SemaphoreType.DMA,
        ],
        compiler_params=pltpu.CompilerParams(needs_layout_passes=False),
    )
    def sc_kernel(enc_hbm, inds_hbm, out_hbm, idx_v, rows_a, rows_b,
                  sem_a, sem_b):
        wid = lax.axis_index("s") * _NUM_CORES + lax.axis_index("c")
        local0 = wid * span
        pltpu.sync_copy(inds_hbm.at[pl.ds(k * rows + local0, span)], idx_v)

        def gather_pair(j2, carry):
            off_a = (2 * j2) * CH
            off_b = off_a + CH
            da = pltpu.async_copy(enc_hbm.at[idx_v.at[pl.ds(off_a, CH)]],
                                  rows_a, sem_a)
            db = pltpu.async_copy(enc_hbm.at[idx_v.at[pl.ds(off_b, CH)]],
                                  rows_b, sem_b)
            da.wait()
            pltpu.sync_copy(rows_a, out_hbm.at[pl.ds(local0 + off_a, CH)])
            db.wait()
            pltpu.sync_copy(rows_b, out_hbm.at[pl.ds(local0 + off_b, CH)])
            return carry

        lax.fori_loop(0, span // (2 * CH), gather_pair, jnp.int32(0))

    return sc_kernel(enc_flat, inds)


def _tc_postnet_chunk(gk, pe, pitch_k, beats_k, W2, wp, dvec, cvec,
                      prev, k, B, T, D):
    nb = B // _NCHUNK

    def body(*refs):
        g_ref, pe_ref, p_ref, bt_ref, w2_ref, wp_ref, dv_ref, cv_ref = refs[:8]
        o_ref = refs[-1]
        pe_t = pe_ref[...]
        x = g_ref[0] + pe_t
        acc = jnp.dot(x, w2_ref[...], preferred_element_type=jnp.float32)
        bt = bt_ref[0].astype(jnp.float32)
        o_ref[0] = (acc - pe_t + p_ref[0] * wp_ref[...]
                    + bt * dv_ref[...] + cv_ref[...])

    in_specs = [
        pl.BlockSpec((1, T, D), lambda bb: (bb, 0, 0)),
        pl.BlockSpec((T, D), lambda bb: (0, 0)),
        pl.BlockSpec((1, T, 1), lambda bb: (bb, 0, 0)),
        pl.BlockSpec((1, T, 1), lambda bb: (bb, 0, 0)),
        pl.BlockSpec((D, D), lambda bb: (0, 0)),
        pl.BlockSpec((1, D), lambda bb: (0, 0)),
        pl.BlockSpec((1, D), lambda bb: (0, 0)),
        pl.BlockSpec((1, D), lambda bb: (0, 0)),
    ]
    args = [gk.reshape(nb, T, D), pe, pitch_k, beats_k, W2, wp, dvec, cvec]
    aliases = {}
    if prev is not None:
        in_specs.append(pl.BlockSpec(memory_space=pl.ANY))
        args.append(prev)
        aliases = {8: 0}

    return pl.pallas_call(
        body,
        grid=(nb,),
        in_specs=in_specs,
        out_specs=pl.BlockSpec((1, T, D), lambda bb, _k=k: (_k * nb + bb, 0, 0)),
        out_shape=jax.ShapeDtypeStruct((B, T, D), jnp.float32),
        input_output_aliases=aliases,
    )(*args)


def kernel(encoder_out, align_phone, text_phone, pitch, beats,
           W_pitch, b_pitch, W_pos, b_pos, emb_beats):
    del text_phone
    B, T, D = encoder_out.shape
    enc_flat = encoder_out.reshape(B * T, D)

    align2d = align_phone.reshape(B * 32, 128)
    alignm1 = jnp.concatenate(
        [align_phone[:, :1], align_phone[:, :-1]], axis=1).reshape(B * 32, 128)
    inds = _tc_inds(align2d, alignm1, B, T)

    pe = _positional_encoding(D, T)
    W2 = W_pos.T + jnp.eye(D, dtype=W_pos.dtype)
    wp = W_pitch.reshape(1, D)
    dvec = (emb_beats[1] - emb_beats[0]).reshape(1, D)
    cvec = (b_pos + b_pitch + emb_beats[0]).reshape(1, D)

    nb = B // _NCHUNK
    gs = [_sc_gather_chunk(enc_flat, inds, k, B, T, D)
          for k in range(_NCHUNK)]
    out = None
    for k in range(_NCHUNK):
        pitch_k = lax.slice_in_dim(pitch, k * nb, (k + 1) * nb, axis=0)
        beats_k = lax.slice_in_dim(beats, k * nb, (k + 1) * nb, axis=0)
        out = _tc_postnet_chunk(gs[k], pe, pitch_k, beats_k, W2, wp, dvec,
                                cvec, out, k, B, T, D)
    return out


# speedup vs baseline: 1.0900x; 1.0041x over previous
import functools
import math

import jax
import jax.numpy as jnp
import numpy as np
from jax import lax
from jax.experimental import pallas as pl
from jax.experimental.pallas import tpu as pltpu
from jax.experimental.pallas import tpu_sc as plsc

_NUM_CORES = 2
_NUM_SUBCORES = 16
_NCHUNK = 1


def _positional_encoding(d_model, length):
    position = np.arange(length, dtype=np.float32)[:, None]
    div_term = np.exp(
        np.arange(0, d_model, 2, dtype=np.float32) * (-math.log(10000.0) / d_model))
    pe = np.zeros((length, d_model), dtype=np.float32)
    pe[:, 0::2] = np.sin(position * div_term)
    pe[:, 1::2] = np.cos(position * div_term)
    return jnp.asarray(pe)


def _tc_inds(align2d, alignm1, B, T):
    R = B * 32
    U = jnp.asarray(np.triu(np.ones((128, 128), np.float32)))
    O = jnp.asarray(np.ones((128, 128), np.float32))
    Lb = jnp.asarray(np.kron(np.eye(B, dtype=np.float32),
                             np.tril(np.ones((32, 32), np.float32), -1)))
    base = jnp.asarray(
        np.repeat(np.arange(B, dtype=np.int32) * T, 32)[:, None]
        * np.ones((1, 128), np.int32))

    def body(a_ref, p_ref, u_ref, o_ref, l_ref, b_ref, out_ref):
        flags = jnp.where(a_ref[...] != p_ref[...], 1.0, 0.0)
        s1 = jnp.dot(flags, u_ref[...], preferred_element_type=jnp.float32)
        rs = jnp.dot(flags, o_ref[...], preferred_element_type=jnp.float32)
        off = jnp.dot(l_ref[...], rs, preferred_element_type=jnp.float32)
        out_ref[...] = (s1 + off).astype(jnp.int32) + b_ref[...]

    out = pl.pallas_call(
        body,
        out_shape=jax.ShapeDtypeStruct((R, 128), jnp.int32),
    )(align2d, alignm1, U, O, Lb, base)
    return out.reshape(B * T)


def _sc_gather_chunk(enc_flat, inds, k, B, T, D):
    NW = _NUM_CORES * _NUM_SUBCORES
    rows = (B // _NCHUNK) * T
    span = rows // NW
    CH = 128

    mesh = plsc.VectorSubcoreMesh(core_axis_name="c", subcore_axis_name="s")

    @functools.partial(
        pl.kernel,
        out_type=jax.ShapeDtypeStruct((rows, D), jnp.float32),
        mesh=mesh,
        scratch_types=[
            pltpu.VMEM((span,), jnp.int32),
            pltpu.VMEM((CH, D), jnp.float32),
            pltpu.VMEM((CH, D), jnp.float32),
            pltpu.VMEM((CH, D), jnp.float32),
            pltpu.SemaphoreType.DMA,
            pltpu.SemaphoreType.DMA,
            pltpu.SemaphoreType.DMA,
            pltpu.SemaphoreType.DMA,
            pltpu.SemaphoreType.DMA,
            pltpu.SemaphoreType.DMA,
        ],
        compiler_params=pltpu.CompilerParams(needs_layout_passes=False),
    )
    def sc_kernel(enc_hbm, inds_hbm, out_hbm, idx_v, buf0, buf1, buf2,
                  gs0, gs1, gs2, ss0, ss1, ss2):
        wid = lax.axis_index("s") * _NUM_CORES + lax.axis_index("c")
        local0 = wid * span
        pltpu.sync_copy(inds_hbm.at[pl.ds(k * rows + local0, span)], idx_v)

        bufs = (buf0, buf1, buf2)
        gsems = (gs0, gs1, gs2)
        ssems = (ss0, ss1, ss2)
        n = span // CH
        gd = [None] * n
        sd = [None] * n
        for j in range(n):
            r = j % 3
            if j >= 3:
                sd[j - 3].wait()
            gd[j] = pltpu.async_copy(
                enc_hbm.at[idx_v.at[pl.ds(j * CH, CH)]], bufs[r], gsems[r])
            if j >= 1:
                p = (j - 1) % 3
                gd[j - 1].wait()
                sd[j - 1] = pltpu.async_copy(
                    bufs[p], out_hbm.at[pl.ds(local0 + (j - 1) * CH, CH)],
                    ssems[p])
        gd[n - 1].wait()
        sd[n - 1] = pltpu.async_copy(
            bufs[(n - 1) % 3],
            out_hbm.at[pl.ds(local0 + (n - 1) * CH, CH)], ssems[(n - 1) % 3])
        for j in range(n - 3, n):
            sd[j].wait()

    return sc_kernel(enc_flat, inds)


def _tc_postnet_chunk(gk, pe, pitch_k, beats_k, W2, wp, dvec, cvec,
                      prev, k, B, T, D):
    nb = B // _NCHUNK

    def body(*refs):
        g_ref, pe_ref, p_ref, bt_ref, w2_ref, wp_ref, dv_ref, cv_ref = refs[:8]
        o_ref = refs[-1]
        pe_t = pe_ref[...]
        x = g_ref[0] + pe_t
        acc = jnp.dot(x, w2_ref[...], preferred_element_type=jnp.float32)
        bt = bt_ref[0].astype(jnp.float32)
        o_ref[0] = (acc - pe_t + p_ref[0] * wp_ref[...]
                    + bt * dv_ref[...] + cv_ref[...])

    in_specs = [
        pl.BlockSpec((1, T, D), lambda bb: (bb, 0, 0)),
        pl.BlockSpec((T, D), lambda bb: (0, 0)),
        pl.BlockSpec((1, T, 1), lambda bb: (bb, 0, 0)),
        pl.BlockSpec((1, T, 1), lambda bb: (bb, 0, 0)),
        pl.BlockSpec((D, D), lambda bb: (0, 0)),
        pl.BlockSpec((1, D), lambda bb: (0, 0)),
        pl.BlockSpec((1, D), lambda bb: (0, 0)),
        pl.BlockSpec((1, D), lambda bb: (0, 0)),
    ]
    args = [gk.reshape(nb, T, D), pe, pitch_k, beats_k, W2, wp, dvec, cvec]
    aliases = {}
    if prev is not None:
        in_specs.append(pl.BlockSpec(memory_space=pl.ANY))
        args.append(prev)
        aliases = {8: 0}

    return pl.pallas_call(
        body,
        grid=(nb,),
        in_specs=in_specs,
        out_specs=pl.BlockSpec((1, T, D), lambda bb, _k=k: (_k * nb + bb, 0, 0)),
        out_shape=jax.ShapeDtypeStruct((B, T, D), jnp.float32),
        input_output_aliases=aliases,
    )(*args)


def kernel(encoder_out, align_phone, text_phone, pitch, beats,
           W_pitch, b_pitch, W_pos, b_pos, emb_beats):
    del text_phone
    B, T, D = encoder_out.shape
    enc_flat = encoder_out.reshape(B * T, D)

    align2d = align_phone.reshape(B * 32, 128)
    alignm1 = jnp.concatenate(
        [align_phone[:, :1], align_phone[:, :-1]], axis=1).reshape(B * 32, 128)
    inds = _tc_inds(align2d, alignm1, B, T)

    pe = _positional_encoding(D, T)
    W2 = W_pos.T + jnp.eye(D, dtype=W_pos.dtype)
    wp = W_pitch.reshape(1, D)
    dvec = (emb_beats[1] - emb_beats[0]).reshape(1, D)
    cvec = (b_pos + b_pitch + emb_beats[0]).reshape(1, D)

    nb = B // _NCHUNK
    gs = [_sc_gather_chunk(enc_flat, inds, k, B, T, D)
          for k in range(_NCHUNK)]
    out = None
    for k in range(_NCHUNK):
        pitch_k = lax.slice_in_dim(pitch, k * nb, (k + 1) * nb, axis=0)
        beats_k = lax.slice_in_dim(beats, k * nb, (k + 1) * nb, axis=0)
        out = _tc_postnet_chunk(gs[k], pe, pitch_k, beats_k, W2, wp, dvec,
                                cvec, out, k, B, T, D)
    return out
